# baseline (speedup 1.0000x reference)
import functools

import jax
import jax.numpy as jnp
from jax import lax
from jax.experimental import pallas as pl
from jax.experimental.pallas import tpu as pltpu
from jax.experimental.pallas import tpu_sc as plsc

N = 50000
L = 4
B = N // L
Bp = 12544
Np = L * Bp
NG = 8
CH = 128
NW = 32
S_DEG = Np + 10 * Bp
DEGW = 8

f32 = jnp.float32
i32 = jnp.int32


def _cdiv(a, b):
    return (a + b - 1) // b


def _chunks(total, step):
    out, o = [], 0
    while o < total:
        n = min(step, total - o)
        out.append((o, n))
        o += n
    return out


@functools.lru_cache(maxsize=None)
def _make_scatter(A, nseg, cseg, npass, chp=CH):
    colw = 64 // npass
    rpt = A // 16
    mesh = plsc.VectorSubcoreMesh(core_axis_name="c", subcore_axis_name="s")

    def body(*refs):
        tables = refs[:npass]
        sidx, didx, zeros, out = refs[npass:npass + 4]
        siv, div, rv0, zb, acc, sg0 = refs[npass + 4:]
        c = lax.axis_index("c")
        s = lax.axis_index("s")
        w = c * 16 + s
        r0 = s * rpt
        pltpu.sync_copy(zeros, zb)
        for p in range(npass):
            tbl = tables[p]
            for (o, n) in _chunks(rpt, CH):
                pltpu.sync_copy(zb.at[pl.ds(0, n)], acc.at[pl.ds(r0 + o, n)])
            plsc.subcore_barrier()
            for seg in range(nseg):
                pltpu.sync_copy(sidx.at[w, pl.ds(seg * cseg, cseg)], siv)
                pltpu.sync_copy(didx.at[w, pl.ds(seg * cseg, cseg)], div)

                def step(j, carry):
                    pltpu.async_copy(tbl.at[siv.at[j]], rv0, sg0).wait()
                    pltpu.sync_copy(rv0, acc.at[div.at[j]], add=True)
                    return carry

                lax.fori_loop(0, cseg, step, 0)
            plsc.subcore_barrier()
            for (o, n) in _chunks(rpt, CH):
                pltpu.sync_copy(acc.at[pl.ds(r0 + o, n)], rv0.at[pl.ds(0, n)])
                pltpu.sync_copy(rv0.at[pl.ds(0, n)], out.at[c, p, pl.ds(r0 + o, n)])

    return pl.kernel(
        body,
        out_type=jax.ShapeDtypeStruct((2, npass, A, colw), f32),
        mesh=mesh,
        compiler_params=pltpu.CompilerParams(use_tc_tiling_on_sc=False),
        scratch_types=[
            pltpu.VMEM((cseg, chp), i32),
            pltpu.VMEM((cseg, chp), i32),
            pltpu.VMEM((chp, colw), f32),
            pltpu.VMEM((CH, colw), f32),
            pltpu.VMEM_SHARED((A, colw), f32),
            pltpu.SemaphoreType.DMA,
        ],
    )


@functools.lru_cache(maxsize=None)
def _make_degree(nseg, cseg, chp=CH):
    rpt = S_DEG // 16
    mesh = plsc.VectorSubcoreMesh(core_axis_name="c", subcore_axis_name="s")

    def body(didx, ones_h, zeros, out, div, ov, zb, acc, sd):
        c = lax.axis_index("c")
        s = lax.axis_index("s")
        w = c * 16 + s
        r0 = s * rpt
        pltpu.sync_copy(ones_h, ov)
        pltpu.sync_copy(zeros, zb)
        for (o, n) in _chunks(rpt, 1024):
            pltpu.sync_copy(zb.at[pl.ds(0, n)], acc.at[pl.ds(r0 + o, n)])
        plsc.subcore_barrier()
        for seg in range(nseg):
            pltpu.sync_copy(didx.at[w, pl.ds(seg * cseg, cseg)], div)

            def step(j, carry):
                pltpu.async_copy(ov, acc.at[div.at[j]], sd, add=True)

                @pl.when(j >= 8)
                def _():
                    pltpu.make_async_copy(ov, acc.at[div.at[0]], sd).wait()
                return carry

            lax.fori_loop(0, cseg, step, 0)

            def drain(j, carry):
                pltpu.make_async_copy(ov, acc.at[div.at[0]], sd).wait()
                return carry

            lax.fori_loop(0, 8, drain, 0)
        plsc.subcore_barrier()
        for (o, n) in _chunks(rpt, 1024):
            pltpu.sync_copy(acc.at[pl.ds(r0 + o, n)], zb.at[pl.ds(0, n)])
            pltpu.sync_copy(zb.at[pl.ds(0, n)], out.at[c, pl.ds(r0 + o, n)])

    return pl.kernel(
        body,
        out_type=jax.ShapeDtypeStruct((2, S_DEG, DEGW), f32),
        mesh=mesh,
        compiler_params=pltpu.CompilerParams(use_tc_tiling_on_sc=False),
        scratch_types=[
            pltpu.VMEM((cseg, chp), i32),
            pltpu.VMEM((chp, DEGW), f32),
            pltpu.VMEM((1024, DEGW), f32),
            pltpu.VMEM_SHARED((S_DEG, DEGW), f32),
            pltpu.SemaphoreType.DMA,
        ],
    )


def _shard_idx(idx, fill, C, chp=CH):
    ep = NW * C * chp
    idx = jnp.pad(idx.astype(i32), (0, ep - idx.shape[0]), constant_values=fill)
    return idx.reshape(NW, C, chp)


def _dinv_body(c_ref, o_ref):
    t = 1.0 + c_ref[0, :, 0:1] + c_ref[1, :, 0:1]
    o_ref[...] = jnp.broadcast_to(lax.rsqrt(t), o_ref.shape)


def _mm_body(x_ref, w_ref, o_ref):
    o_ref[...] = jnp.dot(x_ref[...], w_ref[...], preferred_element_type=f32)


def _prep_scaled_body(x_ref, w_ref, sp_ref, d_ref, o_ref):
    h = jnp.dot(x_ref[...], w_ref[...], preferred_element_type=f32) + sp_ref[...]
    o_ref[...] = h * d_ref[...]


def _prep_scaled_nosp_body(x_ref, w_ref, d_ref, o_ref):
    h = jnp.dot(x_ref[...], w_ref[...], preferred_element_type=f32)
    o_ref[...] = h * d_ref[...]


def _prep_plain_body(x_ref, w_ref, sp_ref, o_ref):
    o_ref[...] = jnp.dot(x_ref[...], w_ref[...], preferred_element_type=f32) + sp_ref[...]


def _prep_pair_body(xa_ref, xb_ref, w_ref, spa_ref, spb_ref, oa_ref, ob_ref):
    oa_ref[...] = jnp.dot(xa_ref[...], w_ref[...], preferred_element_type=f32) + spa_ref[...]
    ob_ref[...] = jnp.dot(xb_ref[...], w_ref[...], preferred_element_type=f32) + spb_ref[...]


def _ssum(s_ref):
    npass = s_ref.shape[1]
    parts = [s_ref[0, p] + s_ref[1, p] for p in range(npass)]
    return parts[0] if npass == 1 else jnp.concatenate(parts, axis=1)


def _fin_inner_body(g_ref, s_ref, d_ref, b_ref, o_ref):
    o_ref[...] = b_ref[...] + d_ref[...] * (g_ref[...] + _ssum(s_ref))


def _fin_cross_body(h_ref, s_ref, d_ref, b_ref, o_ref):
    d = d_ref[...]
    o_ref[...] = b_ref[...] + d * (d * h_ref[...] + _ssum(s_ref))


def _relu4_body(a_ref, b_ref, c_ref, d_ref, oa, ob, oc, od):
    oa[...] = jnp.maximum(a_ref[...], 0.0)
    ob[...] = jnp.maximum(b_ref[...], 0.0)
    oc[...] = jnp.maximum(c_ref[...], 0.0)
    od[...] = jnp.maximum(d_ref[...], 0.0)


def _pool_body(x_ref, b_ref, wl_ref, bl_ref, o_ref):
    i = pl.program_id(0)
    y = jnp.dot(x_ref[...], wl_ref[...], preferred_element_type=f32)
    m = (b_ref[...] == lax.broadcasted_iota(i32, (x_ref.shape[0], NG), 1))
    part = lax.dot_general(m.astype(f32), y, (((0,), (0,)), ((), ())),
                           preferred_element_type=f32)

    @pl.when(i == 0)
    def _():
        o_ref[...] = bl_ref[...] + part

    @pl.when(i != 0)
    def _():
        o_ref[...] += part


def _sds(shape):
    return jax.ShapeDtypeStruct(shape, f32)


def _dinv_dense(cnt):
    nblk = S_DEG // Bp
    return pl.pallas_call(
        _dinv_body,
        grid=(nblk,),
        in_specs=[pl.BlockSpec((2, Bp, DEGW), lambda i: (0, i, 0))],
        out_specs=pl.BlockSpec((Bp, 64), lambda i: (i, 0)),
        out_shape=_sds((S_DEG, 64)),
    )(cnt)


def _mm(xmat, w):
    return pl.pallas_call(_mm_body, out_shape=_sds((xmat.shape[0], 64)))(xmat, w)


def _prep_scaled(xblk, w, sp, d):
    return pl.pallas_call(_prep_scaled_body, out_shape=_sds((Bp, 64)))(xblk, w, sp, d)


def _prep_plain(xblk, w, sp):
    return pl.pallas_call(_prep_plain_body, out_shape=_sds((Bp, 64)))(xblk, w, sp)


def _prep_pair(xa, xb, w, spa, spb):
    return pl.pallas_call(
        _prep_pair_body, out_shape=[_sds((Bp, 64))] * 2)(xa, xb, w, spa, spb)


def _fin_inner_blk(g, spart, d, bvec):
    return pl.pallas_call(_fin_inner_body, out_shape=_sds((Bp, 64)))(g, spart, d, bvec)


def _fin_cross_blk(h, spart, d, bvec):
    return pl.pallas_call(_fin_cross_body, out_shape=_sds((Bp, 64)))(h, spart, d, bvec)


def _prep_main(xfull, w, d):
    return pl.pallas_call(
        _prep_scaled_nosp_body,
        grid=(L,),
        in_specs=[pl.BlockSpec((Bp, 64), lambda i: (i, 0)),
                  pl.BlockSpec((64, 64), lambda i: (0, 0)),
                  pl.BlockSpec((Bp, 64), lambda i: (i, 0))],
        out_specs=pl.BlockSpec((Bp, 64), lambda i: (i, 0)),
        out_shape=_sds((Np, 64)),
    )(xfull, w, d)


def _fin_main(g, spart, d, bvec):
    R = Np // 32
    return pl.pallas_call(
        _fin_inner_body,
        grid=(32,),
        in_specs=[pl.BlockSpec((R, 64), lambda i: (i, 0)),
                  pl.BlockSpec((2, 4, R, 16), lambda i: (0, 0, i, 0)),
                  pl.BlockSpec((R, 64), lambda i: (i, 0)),
                  pl.BlockSpec((1, 64), lambda i: (0, 0))],
        out_specs=pl.BlockSpec((R, 64), lambda i: (i, 0)),
        out_shape=_sds((Np, 64)),
    )(g, spart, d, bvec)


def _relu4(blocks):
    outs = pl.pallas_call(
        _relu4_body,
        out_shape=[_sds((Bp, 64))] * 4,
    )(*blocks)
    return list(outs)


def _pool(xfull, batch_p, wl, bl):
    R = Np // 8
    return pl.pallas_call(
        _pool_body,
        grid=(8,),
        in_specs=[pl.BlockSpec((R, 64), lambda i: (i, 0)),
                  pl.BlockSpec((R, 1), lambda i: (i, 0)),
                  pl.BlockSpec((64, 1), lambda i: (0, 0)),
                  pl.BlockSpec((1, 1), lambda i: (0, 0))],
        out_specs=pl.BlockSpec((NG, 1), lambda i: (0, 0)),
        out_shape=_sds((NG, 1)),
    )(xfull, batch_p, wl, bl)


def kernel(x, feature_mtx_static, edge_index, inner_edges, forward_edges,
           backward_edges, batch, W_up, b_up, W_in, b_in, W_fw, b_fw,
           W_bw, b_bw, W_lin, b_lin):
    def padrows(a):
        return jnp.pad(a, ((0, Bp - B), (0, 0)))

    xb = [padrows(x[l * B:(l + 1) * B]) for l in range(L)]
    stp = jnp.concatenate(
        [padrows(feature_mtx_static[l * B:(l + 1) * B]) for l in range(L)], 0)

    remap = lambda v: (v + 44 * (v // B)).astype(i32)
    src_m = remap(edge_index[0])
    dst_m = remap(edge_index[1])

    CHP = 512
    C_IN = 13
    C_X = 7
    C_M = 49
    C_D = 136

    in_s = [_shard_idx(inner_edges[l, 0], 0, C_IN, CHP) for l in range(L)]
    in_d = [_shard_idx(inner_edges[l, 1], Bp - 1, C_IN, CHP) for l in range(L)]
    fw_s = [_shard_idx(forward_edges[l, 0], 0, C_X, CHP) for l in range(L - 1)]
    fw_d = [_shard_idx(forward_edges[l, 1], Bp - 1, C_X, CHP) for l in range(L - 1)]
    bw_s = [_shard_idx(backward_edges[l, 0], 0, C_X, CHP) for l in range(L - 1)]
    bw_d = [_shard_idx(backward_edges[l, 1], Bp - 1, C_X, CHP) for l in range(L - 1)]
    m_s = _shard_idx(src_m, 0, C_M, CHP)
    m_d = _shard_idx(dst_m, Np - 1, C_M, CHP)

    off_in = [Np + l * Bp for l in range(L)]
    off_fw = [Np + (4 + l) * Bp for l in range(L - 1)]
    off_bw = [Np + (7 + l) * Bp for l in range(L - 1)]
    deg_dst = jnp.concatenate(
        [dst_m]
        + [inner_edges[l, 1].astype(i32) + off_in[l] for l in range(L)]
        + [forward_edges[l, 1].astype(i32) + off_fw[l] for l in range(L - 1)]
        + [backward_edges[l, 1].astype(i32) + off_bw[l] for l in range(L - 1)])
    deg_sh = _shard_idx(deg_dst, S_DEG - 1, C_D, CHP)

    zeros_blk = jnp.zeros((CH, 64), f32)
    zeros_m16 = jnp.zeros((CH, 16), f32)
    zeros_deg = jnp.zeros((1024, DEGW), f32)
    ones_deg = jnp.ones((CHP, DEGW), f32)
    bu = b_up.reshape(1, 64)
    bi = b_in.reshape(1, 64)
    bf = b_fw.reshape(1, 64)
    bb = b_bw.reshape(1, 64)
    bl = b_lin.reshape(1, 1)

    cnt = _make_degree(17, 8, CHP)(deg_sh, ones_deg, zeros_deg)
    dall = _dinv_dense(cnt)
    d_m = dall[0:Np]
    d_in = [dall[off_in[l]:off_in[l] + Bp] for l in range(L)]
    d_fw = [dall[off_fw[l]:off_fw[l] + Bp] for l in range(L - 1)]
    d_bw = [dall[off_bw[l]:off_bw[l] + Bp] for l in range(L - 1)]

    sp_in = _mm(stp, W_in[64:])
    sp_fw = _mm(stp, W_fw[64:])
    sp_bw = _mm(stp, W_bw[64:])
    spb_in = [sp_in[l * Bp:(l + 1) * Bp] for l in range(L)]
    spb_fw = [sp_fw[l * Bp:(l + 1) * Bp] for l in range(L)]
    spb_bw = [sp_bw[l * Bp:(l + 1) * Bp] for l in range(L)]
    Wd_in, Wd_fw, Wd_bw = W_in[:64], W_fw[:64], W_bw[:64]

    scat_blk = _make_scatter(Bp, 1, C_IN, 1, CHP)
    scat_x = _make_scatter(Bp, 1, C_X, 1, CHP)
    scat_main = _make_scatter(Np, 1, C_M, 4, CHP)

    xfull = jnp.concatenate(xb, 0)
    g = _prep_main(xfull, W_up, d_m)
    s = scat_main(g[:, 0:16], g[:, 16:32], g[:, 32:48], g[:, 48:64],
                  m_s, m_d, zeros_m16)
    xfull = _fin_main(g, s, d_m, bu)
    xb = [xfull[l * Bp:(l + 1) * Bp] for l in range(L)]

    def inner(xb, l):
        g = _prep_scaled(xb[l], Wd_in, spb_in[l], d_in[l])
        spart = scat_blk(g, in_s[l], in_d[l], zeros_blk)
        xb[l] = _fin_inner_blk(g, spart, d_in[l], bi)

    def fwd(xb, l):
        h_src, h_dst = _prep_pair(xb[l], xb[l + 1], Wd_fw, spb_fw[l], spb_fw[l + 1])
        spart = scat_x(h_src, fw_s[l], fw_d[l], zeros_blk)
        xb[l + 1] = _fin_cross_blk(h_dst, spart, d_fw[l], bf)

    def bwd(xb, l):
        h_src, h_dst = _prep_pair(xb[l], xb[l - 1], Wd_bw, spb_bw[l], spb_bw[l - 1])
        spart = scat_x(h_src, bw_s[l - 1], bw_d[l - 1], zeros_blk)
        xb[l - 1] = _fin_cross_blk(h_dst, spart, d_bw[l - 1], bb)

    for _ in range(2):
        for l in range(L):
            inner(xb, l)
            if l < L - 1:
                fwd(xb, l)
        xb = _relu4(xb)
        for l in range(L - 1, 0, -1):
            bwd(xb, l)
            inner(xb, l - 1)
        xb = _relu4(xb)

    xfull = jnp.concatenate(xb, 0)
    batch_p = jnp.concatenate(
        [jnp.pad(batch[l * B:(l + 1) * B].astype(i32), (0, Bp - B),
                 constant_values=NG) for l in range(L)]).reshape(Np, 1)
    return _pool(xfull, batch_p, W_lin, bl)


# speedup vs baseline: 1.3995x; 1.3995x over previous
import functools

import jax
import jax.numpy as jnp
from jax import lax
from jax.experimental import pallas as pl
from jax.experimental.pallas import tpu as pltpu
from jax.experimental.pallas import tpu_sc as plsc

N = 50000
L = 4
B = N // L
Bp = 12544
Np = L * Bp
NG = 8
CH = 128
NW = 32
S_DEG = Np + 10 * Bp
DEGW = 8

f32 = jnp.float32
i32 = jnp.int32


def _cdiv(a, b):
    return (a + b - 1) // b


def _chunks(total, step):
    out, o = [], 0
    while o < total:
        n = min(step, total - o)
        out.append((o, n))
        o += n
    return out


@functools.lru_cache(maxsize=None)
def _make_scatter(A, nseg, cseg, npass, chp=CH):
    colw = 64 // npass
    rpt = A // 16
    mesh = plsc.VectorSubcoreMesh(core_axis_name="c", subcore_axis_name="s")

    def body(*refs):
        tables = refs[:npass]
        sidx, didx, zeros, out = refs[npass:npass + 4]
        siv, div, rv0, zb, acc, sg0 = refs[npass + 4:]
        c = lax.axis_index("c")
        s = lax.axis_index("s")
        w = c * 16 + s
        r0 = s * rpt
        pltpu.sync_copy(zeros, zb)
        for p in range(npass):
            tbl = tables[p]
            for (o, n) in _chunks(rpt, CH):
                pltpu.sync_copy(zb.at[pl.ds(0, n)], acc.at[pl.ds(r0 + o, n)])
            plsc.subcore_barrier()
            for seg in range(nseg):
                pltpu.sync_copy(sidx.at[w, pl.ds(seg * cseg, cseg)], siv)
                pltpu.sync_copy(didx.at[w, pl.ds(seg * cseg, cseg)], div)

                def step(j, carry):
                    pltpu.async_copy(tbl.at[siv.at[j]], rv0, sg0).wait()
                    pltpu.sync_copy(rv0, acc.at[div.at[j]], add=True)
                    return carry

                lax.fori_loop(0, cseg, step, 0)
            plsc.subcore_barrier()
            for (o, n) in _chunks(rpt, CH):
                pltpu.sync_copy(acc.at[pl.ds(r0 + o, n)], rv0.at[pl.ds(0, n)])
                pltpu.sync_copy(rv0.at[pl.ds(0, n)], out.at[c, p, pl.ds(r0 + o, n)])

    return pl.kernel(
        body,
        out_type=jax.ShapeDtypeStruct((2, npass, A, colw), f32),
        mesh=mesh,
        compiler_params=pltpu.CompilerParams(use_tc_tiling_on_sc=False),
        scratch_types=[
            pltpu.VMEM((cseg, chp), i32),
            pltpu.VMEM((cseg, chp), i32),
            pltpu.VMEM((chp, colw), f32),
            pltpu.VMEM((CH, colw), f32),
            pltpu.VMEM_SHARED((A, colw), f32),
            pltpu.SemaphoreType.DMA,
        ],
    )


@functools.lru_cache(maxsize=None)
def _make_degree(nseg, cseg, chp=CH):
    rpt = S_DEG // 16
    mesh = plsc.VectorSubcoreMesh(core_axis_name="c", subcore_axis_name="s")

    def body(didx, ones_h, zeros, out, div, ov, zb, acc, sd):
        c = lax.axis_index("c")
        s = lax.axis_index("s")
        w = c * 16 + s
        r0 = s * rpt
        pltpu.sync_copy(ones_h, ov)
        pltpu.sync_copy(zeros, zb)
        for (o, n) in _chunks(rpt, 1024):
            pltpu.sync_copy(zb.at[pl.ds(0, n)], acc.at[pl.ds(r0 + o, n)])
        plsc.subcore_barrier()
        for seg in range(nseg):
            pltpu.sync_copy(didx.at[w, pl.ds(seg * cseg, cseg)], div)

            def step(j, carry):
                pltpu.async_copy(ov, acc.at[div.at[j]], sd, add=True)

                @pl.when(j >= 8)
                def _():
                    pltpu.make_async_copy(ov, acc.at[div.at[0]], sd).wait()
                return carry

            lax.fori_loop(0, cseg, step, 0)

            def drain(j, carry):
                pltpu.make_async_copy(ov, acc.at[div.at[0]], sd).wait()
                return carry

            lax.fori_loop(0, 8, drain, 0)
        plsc.subcore_barrier()
        for (o, n) in _chunks(rpt, 1024):
            pltpu.sync_copy(acc.at[pl.ds(r0 + o, n)], zb.at[pl.ds(0, n)])
            pltpu.sync_copy(zb.at[pl.ds(0, n)], out.at[c, pl.ds(r0 + o, n)])

    return pl.kernel(
        body,
        out_type=jax.ShapeDtypeStruct((2, S_DEG, DEGW), f32),
        mesh=mesh,
        compiler_params=pltpu.CompilerParams(use_tc_tiling_on_sc=False),
        scratch_types=[
            pltpu.VMEM((cseg, chp), i32),
            pltpu.VMEM((chp, DEGW), f32),
            pltpu.VMEM((1024, DEGW), f32),
            pltpu.VMEM_SHARED((S_DEG, DEGW), f32),
            pltpu.SemaphoreType.DMA,
        ],
    )


def _shard_idx(idx, fill, C, chp=CH):
    ep = NW * C * chp
    idx = jnp.pad(idx.astype(i32), (0, ep - idx.shape[0]), constant_values=fill)
    return idx.reshape(NW, C, chp)


def _dinv_body(c_ref, o_ref):
    t = 1.0 + c_ref[0, :, 0:1] + c_ref[1, :, 0:1]
    o_ref[...] = jnp.broadcast_to(lax.rsqrt(t), o_ref.shape)


def _mm_body(x_ref, w_ref, o_ref):
    o_ref[...] = jnp.dot(x_ref[...], w_ref[...], preferred_element_type=f32)


def _prep_scaled_body(x_ref, w_ref, sp_ref, d_ref, o_ref):
    h = jnp.dot(x_ref[...], w_ref[...], preferred_element_type=f32) + sp_ref[...]
    o_ref[...] = h * d_ref[...]


def _prep_scaled_nosp_body(x_ref, w_ref, d_ref, o_ref):
    h = jnp.dot(x_ref[...], w_ref[...], preferred_element_type=f32)
    o_ref[...] = h * d_ref[...]


def _prep_plain_body(x_ref, w_ref, sp_ref, o_ref):
    o_ref[...] = jnp.dot(x_ref[...], w_ref[...], preferred_element_type=f32) + sp_ref[...]


def _prep_pair_body(xa_ref, xb_ref, w_ref, spa_ref, spb_ref, oa_ref, ob_ref):
    oa_ref[...] = jnp.dot(xa_ref[...], w_ref[...], preferred_element_type=f32) + spa_ref[...]
    ob_ref[...] = jnp.dot(xb_ref[...], w_ref[...], preferred_element_type=f32) + spb_ref[...]


def _ssum(s_ref):
    npass = s_ref.shape[1]
    parts = [s_ref[0, p] + s_ref[1, p] for p in range(npass)]
    return parts[0] if npass == 1 else jnp.concatenate(parts, axis=1)


def _fin_inner_body(g_ref, s_ref, d_ref, b_ref, o_ref):
    o_ref[...] = b_ref[...] + d_ref[...] * (g_ref[...] + _ssum(s_ref))


def _fin_cross_body(h_ref, s_ref, d_ref, b_ref, o_ref):
    d = d_ref[...]
    o_ref[...] = b_ref[...] + d * (d * h_ref[...] + _ssum(s_ref))


def _relu4_body(a_ref, b_ref, c_ref, d_ref, oa, ob, oc, od):
    oa[...] = jnp.maximum(a_ref[...], 0.0)
    ob[...] = jnp.maximum(b_ref[...], 0.0)
    oc[...] = jnp.maximum(c_ref[...], 0.0)
    od[...] = jnp.maximum(d_ref[...], 0.0)


def _pool_body(x_ref, b_ref, wl_ref, bl_ref, o_ref):
    i = pl.program_id(0)
    y = jnp.dot(x_ref[...], wl_ref[...], preferred_element_type=f32)
    m = (b_ref[...] == lax.broadcasted_iota(i32, (x_ref.shape[0], NG), 1))
    part = lax.dot_general(m.astype(f32), y, (((0,), (0,)), ((), ())),
                           preferred_element_type=f32)

    @pl.when(i == 0)
    def _():
        o_ref[...] = bl_ref[...] + part

    @pl.when(i != 0)
    def _():
        o_ref[...] += part


def _sds(shape):
    return jax.ShapeDtypeStruct(shape, f32)


def _dinv_dense(cnt):
    nblk = S_DEG // Bp
    return pl.pallas_call(
        _dinv_body,
        grid=(nblk,),
        in_specs=[pl.BlockSpec((2, Bp, DEGW), lambda i: (0, i, 0))],
        out_specs=pl.BlockSpec((Bp, 64), lambda i: (i, 0)),
        out_shape=_sds((S_DEG, 64)),
    )(cnt)


def _mm(xmat, w):
    return pl.pallas_call(_mm_body, out_shape=_sds((xmat.shape[0], 64)))(xmat, w)


def _prep_scaled(xblk, w, sp, d):
    return pl.pallas_call(_prep_scaled_body, out_shape=_sds((Bp, 64)))(xblk, w, sp, d)


def _prep_plain(xblk, w, sp):
    return pl.pallas_call(_prep_plain_body, out_shape=_sds((Bp, 64)))(xblk, w, sp)


def _prep_pair(xa, xb, w, spa, spb):
    return pl.pallas_call(
        _prep_pair_body, out_shape=[_sds((Bp, 64))] * 2)(xa, xb, w, spa, spb)


def _fin_inner_blk(g, spart, d, bvec):
    return pl.pallas_call(_fin_inner_body, out_shape=_sds((Bp, 64)))(g, spart, d, bvec)


def _fin_cross_blk(h, spart, d, bvec):
    return pl.pallas_call(_fin_cross_body, out_shape=_sds((Bp, 64)))(h, spart, d, bvec)


def _prep_main(xfull, w, d):
    return pl.pallas_call(
        _prep_scaled_nosp_body,
        grid=(L,),
        in_specs=[pl.BlockSpec((Bp, 64), lambda i: (i, 0)),
                  pl.BlockSpec((64, 64), lambda i: (0, 0)),
                  pl.BlockSpec((Bp, 64), lambda i: (i, 0))],
        out_specs=pl.BlockSpec((Bp, 64), lambda i: (i, 0)),
        out_shape=_sds((Np, 64)),
    )(xfull, w, d)


def _fin_main(g, spart, d, bvec):
    R = Np // 32
    return pl.pallas_call(
        _fin_inner_body,
        grid=(32,),
        in_specs=[pl.BlockSpec((R, 64), lambda i: (i, 0)),
                  pl.BlockSpec((2, 4, R, 16), lambda i: (0, 0, i, 0)),
                  pl.BlockSpec((R, 64), lambda i: (i, 0)),
                  pl.BlockSpec((1, 64), lambda i: (0, 0))],
        out_specs=pl.BlockSpec((R, 64), lambda i: (i, 0)),
        out_shape=_sds((Np, 64)),
    )(g, spart, d, bvec)


def _relu4(blocks):
    outs = pl.pallas_call(
        _relu4_body,
        out_shape=[_sds((Bp, 64))] * 4,
    )(*blocks)
    return list(outs)


def _pool(xfull, batch_p, wl, bl):
    R = Np // 8
    return pl.pallas_call(
        _pool_body,
        grid=(8,),
        in_specs=[pl.BlockSpec((R, 64), lambda i: (i, 0)),
                  pl.BlockSpec((R, 1), lambda i: (i, 0)),
                  pl.BlockSpec((64, 1), lambda i: (0, 0)),
                  pl.BlockSpec((1, 1), lambda i: (0, 0))],
        out_specs=pl.BlockSpec((NG, 1), lambda i: (0, 0)),
        out_shape=_sds((NG, 1)),
    )(xfull, batch_p, wl, bl)


def kernel(x, feature_mtx_static, edge_index, inner_edges, forward_edges,
           backward_edges, batch, W_up, b_up, W_in, b_in, W_fw, b_fw,
           W_bw, b_bw, W_lin, b_lin):
    def padrows(a):
        return jnp.pad(a, ((0, Bp - B), (0, 0)))

    xb = [padrows(x[l * B:(l + 1) * B]) for l in range(L)]
    stp = jnp.concatenate(
        [padrows(feature_mtx_static[l * B:(l + 1) * B]) for l in range(L)], 0)

    remap = lambda v: (v + 44 * (v // B)).astype(i32)
    src_m = remap(edge_index[0])
    dst_m = remap(edge_index[1])

    CHP = 128
    C_IN = 50
    C_X = 26
    C_M = 200
    C_D = 576

    in_s = [_shard_idx(inner_edges[l, 0], 0, C_IN, CHP) for l in range(L)]
    in_d = [_shard_idx(inner_edges[l, 1], Bp - 1, C_IN, CHP) for l in range(L)]
    fw_s = [_shard_idx(forward_edges[l, 0], 0, C_X, CHP) for l in range(L - 1)]
    fw_d = [_shard_idx(forward_edges[l, 1], Bp - 1, C_X, CHP) for l in range(L - 1)]
    bw_s = [_shard_idx(backward_edges[l, 0], 0, C_X, CHP) for l in range(L - 1)]
    bw_d = [_shard_idx(backward_edges[l, 1], Bp - 1, C_X, CHP) for l in range(L - 1)]
    m_s = _shard_idx(src_m, 0, C_M, CHP)
    m_d = _shard_idx(dst_m, Np - 1, C_M, CHP)

    off_in = [Np + l * Bp for l in range(L)]
    off_fw = [Np + (4 + l) * Bp for l in range(L - 1)]
    off_bw = [Np + (7 + l) * Bp for l in range(L - 1)]
    deg_dst = jnp.concatenate(
        [dst_m]
        + [inner_edges[l, 1].astype(i32) + off_in[l] for l in range(L)]
        + [forward_edges[l, 1].astype(i32) + off_fw[l] for l in range(L - 1)]
        + [backward_edges[l, 1].astype(i32) + off_bw[l] for l in range(L - 1)])
    deg_sh = _shard_idx(deg_dst, S_DEG - 1, C_D, CHP)

    zeros_blk = jnp.zeros((CH, 64), f32)
    zeros_m16 = jnp.zeros((CH, 16), f32)
    zeros_deg = jnp.zeros((1024, DEGW), f32)
    ones_deg = jnp.ones((CHP, DEGW), f32)
    bu = b_up.reshape(1, 64)
    bi = b_in.reshape(1, 64)
    bf = b_fw.reshape(1, 64)
    bb = b_bw.reshape(1, 64)
    bl = b_lin.reshape(1, 1)

    cnt = _make_degree(9, 64, CHP)(deg_sh, ones_deg, zeros_deg)
    dall = _dinv_dense(cnt)
    d_m = dall[0:Np]
    d_in = [dall[off_in[l]:off_in[l] + Bp] for l in range(L)]
    d_fw = [dall[off_fw[l]:off_fw[l] + Bp] for l in range(L - 1)]
    d_bw = [dall[off_bw[l]:off_bw[l] + Bp] for l in range(L - 1)]

    sp_in = _mm(stp, W_in[64:])
    sp_fw = _mm(stp, W_fw[64:])
    sp_bw = _mm(stp, W_bw[64:])
    spb_in = [sp_in[l * Bp:(l + 1) * Bp] for l in range(L)]
    spb_fw = [sp_fw[l * Bp:(l + 1) * Bp] for l in range(L)]
    spb_bw = [sp_bw[l * Bp:(l + 1) * Bp] for l in range(L)]
    Wd_in, Wd_fw, Wd_bw = W_in[:64], W_fw[:64], W_bw[:64]

    scat_blk = _make_scatter(Bp, 1, C_IN, 1, CHP)
    scat_x = _make_scatter(Bp, 1, C_X, 1, CHP)
    scat_main = _make_scatter(Np, 5, 40, 4, CHP)

    xfull = jnp.concatenate(xb, 0)
    g = _prep_main(xfull, W_up, d_m)
    s = scat_main(g[:, 0:16], g[:, 16:32], g[:, 32:48], g[:, 48:64],
                  m_s, m_d, zeros_m16)
    xfull = _fin_main(g, s, d_m, bu)
    xb = [xfull[l * Bp:(l + 1) * Bp] for l in range(L)]

    def inner(xb, l):
        g = _prep_scaled(xb[l], Wd_in, spb_in[l], d_in[l])
        spart = scat_blk(g, in_s[l], in_d[l], zeros_blk)
        xb[l] = _fin_inner_blk(g, spart, d_in[l], bi)

    def fwd(xb, l):
        h_src, h_dst = _prep_pair(xb[l], xb[l + 1], Wd_fw, spb_fw[l], spb_fw[l + 1])
        spart = scat_x(h_src, fw_s[l], fw_d[l], zeros_blk)
        xb[l + 1] = _fin_cross_blk(h_dst, spart, d_fw[l], bf)

    def bwd(xb, l):
        h_src, h_dst = _prep_pair(xb[l], xb[l - 1], Wd_bw, spb_bw[l], spb_bw[l - 1])
        spart = scat_x(h_src, bw_s[l - 1], bw_d[l - 1], zeros_blk)
        xb[l - 1] = _fin_cross_blk(h_dst, spart, d_bw[l - 1], bb)

    for _ in range(2):
        for l in range(L):
            inner(xb, l)
            if l < L - 1:
                fwd(xb, l)
        xb = _relu4(xb)
        for l in range(L - 1, 0, -1):
            bwd(xb, l)
            inner(xb, l - 1)
        xb = _relu4(xb)

    xfull = jnp.concatenate(xb, 0)
    batch_p = jnp.concatenate(
        [jnp.pad(batch[l * B:(l + 1) * B].astype(i32), (0, Bp - B),
                 constant_values=NG) for l in range(L)]).reshape(Np, 1)
    return _pool(xfull, batch_p, W_lin, bl)


# speedup vs baseline: 1.9728x; 1.4097x over previous
import functools

import jax
import jax.numpy as jnp
from jax import lax
from jax.experimental import pallas as pl
from jax.experimental.pallas import tpu as pltpu
from jax.experimental.pallas import tpu_sc as plsc

N = 50000
L = 4
B = N // L
Bp = 12544
Np = L * Bp
NG = 8
CH = 128
NW = 32
S_DEG = Np + 10 * Bp
DEGW = 8

f32 = jnp.float32
i32 = jnp.int32


def _cdiv(a, b):
    return (a + b - 1) // b


def _chunks(total, step):
    out, o = [], 0
    while o < total:
        n = min(step, total - o)
        out.append((o, n))
        o += n
    return out


@functools.lru_cache(maxsize=None)
def _make_scatter(A, nseg, cseg, npass, chp=CH):
    colw = 64 // npass
    rpt = A // 16
    mesh = plsc.VectorSubcoreMesh(core_axis_name="c", subcore_axis_name="s")

    def body(*refs):
        tables = refs[:npass]
        sidx, didx, zeros, out = refs[npass:npass + 4]
        siv, div, rv0, zb, acc, sg0 = refs[npass + 4:]
        c = lax.axis_index("c")
        s = lax.axis_index("s")
        w = c * 16 + s
        r0 = s * rpt
        pltpu.sync_copy(zeros, zb)
        for p in range(npass):
            tbl = tables[p]
            for (o, n) in _chunks(rpt, CH):
                pltpu.sync_copy(zb.at[pl.ds(0, n)], acc.at[pl.ds(r0 + o, n)])
            plsc.subcore_barrier()
            for seg in range(nseg):
                pltpu.sync_copy(sidx.at[w, pl.ds(seg * cseg, cseg)], siv)
                pltpu.sync_copy(didx.at[w, pl.ds(seg * cseg, cseg)], div)

                def step(j, carry):
                    pltpu.async_copy(tbl.at[siv.at[j]], rv0, sg0).wait()
                    pltpu.sync_copy(rv0, acc.at[div.at[j]], add=True)
                    return carry

                lax.fori_loop(0, cseg, step, 0)
            plsc.subcore_barrier()
            for (o, n) in _chunks(rpt, CH):
                pltpu.sync_copy(acc.at[pl.ds(r0 + o, n)], rv0.at[pl.ds(0, n)])
                pltpu.sync_copy(rv0.at[pl.ds(0, n)], out.at[c, p, pl.ds(r0 + o, n)])

    return pl.kernel(
        body,
        out_type=jax.ShapeDtypeStruct((2, npass, A, colw), f32),
        mesh=mesh,
        compiler_params=pltpu.CompilerParams(use_tc_tiling_on_sc=False),
        scratch_types=[
            pltpu.VMEM((cseg, chp), i32),
            pltpu.VMEM((cseg, chp), i32),
            pltpu.VMEM((chp, colw), f32),
            pltpu.VMEM((CH, colw), f32),
            pltpu.VMEM_SHARED((A, colw), f32),
            pltpu.SemaphoreType.DMA,
        ],
    )


@functools.lru_cache(maxsize=None)
def _make_degree(nseg, cseg, chp=CH):
    rpt = S_DEG // 16
    mesh = plsc.VectorSubcoreMesh(core_axis_name="c", subcore_axis_name="s")

    def body(didx, ones_h, zeros, out, div, ov, zb, acc, sd):
        c = lax.axis_index("c")
        s = lax.axis_index("s")
        w = c * 16 + s
        r0 = s * rpt
        pltpu.sync_copy(ones_h, ov)
        pltpu.sync_copy(zeros, zb)
        for (o, n) in _chunks(rpt, 1024):
            pltpu.sync_copy(zb.at[pl.ds(0, n)], acc.at[pl.ds(r0 + o, n)])
        plsc.subcore_barrier()
        for seg in range(nseg):
            pltpu.sync_copy(didx.at[w, pl.ds(seg * cseg, cseg)], div)

            def step(j, carry):
                pltpu.sync_copy(ov, acc.at[div.at[j]], add=True)
                return carry

            lax.fori_loop(0, cseg, step, 0)
        plsc.subcore_barrier()
        for (o, n) in _chunks(rpt, 1024):
            pltpu.sync_copy(acc.at[pl.ds(r0 + o, n)], zb.at[pl.ds(0, n)])
            pltpu.sync_copy(zb.at[pl.ds(0, n)], out.at[c, pl.ds(r0 + o, n)])

    return pl.kernel(
        body,
        out_type=jax.ShapeDtypeStruct((2, S_DEG, DEGW), f32),
        mesh=mesh,
        compiler_params=pltpu.CompilerParams(use_tc_tiling_on_sc=False),
        scratch_types=[
            pltpu.VMEM((cseg, chp), i32),
            pltpu.VMEM((chp, DEGW), f32),
            pltpu.VMEM((1024, DEGW), f32),
            pltpu.VMEM_SHARED((S_DEG, DEGW), f32),
            pltpu.SemaphoreType.DMA,
        ],
    )


def _shard_idx(idx, fill, C, chp=CH):
    ep = NW * C * chp
    idx = jnp.pad(idx.astype(i32), (0, ep - idx.shape[0]), constant_values=fill)
    return idx.reshape(NW, C, chp)


def _dinv_body(c_ref, o_ref):
    t = 1.0 + c_ref[0, :, 0:1] + c_ref[1, :, 0:1]
    o_ref[...] = jnp.broadcast_to(lax.rsqrt(t), o_ref.shape)


def _mm_body(x_ref, w_ref, o_ref):
    o_ref[...] = jnp.dot(x_ref[...], w_ref[...], preferred_element_type=f32)


def _prep_scaled_body(x_ref, w_ref, sp_ref, d_ref, o_ref):
    h = jnp.dot(x_ref[...], w_ref[...], preferred_element_type=f32) + sp_ref[...]
    o_ref[...] = h * d_ref[...]


def _prep_scaled_nosp_body(x_ref, w_ref, d_ref, o_ref):
    h = jnp.dot(x_ref[...], w_ref[...], preferred_element_type=f32)
    o_ref[...] = h * d_ref[...]


def _prep_plain_body(x_ref, w_ref, sp_ref, o_ref):
    o_ref[...] = jnp.dot(x_ref[...], w_ref[...], preferred_element_type=f32) + sp_ref[...]


def _prep_pair_body(xa_ref, xb_ref, w_ref, spa_ref, spb_ref, oa_ref, ob_ref):
    oa_ref[...] = jnp.dot(xa_ref[...], w_ref[...], preferred_element_type=f32) + spa_ref[...]
    ob_ref[...] = jnp.dot(xb_ref[...], w_ref[...], preferred_element_type=f32) + spb_ref[...]


def _ssum(s_ref):
    npass = s_ref.shape[1]
    parts = [s_ref[0, p] + s_ref[1, p] for p in range(npass)]
    return parts[0] if npass == 1 else jnp.concatenate(parts, axis=1)


def _fin_inner_body(g_ref, s_ref, d_ref, b_ref, o_ref):
    o_ref[...] = b_ref[...] + d_ref[...] * (g_ref[...] + _ssum(s_ref))


def _fin_cross_body(h_ref, s_ref, d_ref, b_ref, o_ref):
    d = d_ref[...]
    o_ref[...] = b_ref[...] + d * (d * h_ref[...] + _ssum(s_ref))


def _relu4_body(a_ref, b_ref, c_ref, d_ref, oa, ob, oc, od):
    oa[...] = jnp.maximum(a_ref[...], 0.0)
    ob[...] = jnp.maximum(b_ref[...], 0.0)
    oc[...] = jnp.maximum(c_ref[...], 0.0)
    od[...] = jnp.maximum(d_ref[...], 0.0)


def _pool_body(x_ref, b_ref, wl_ref, bl_ref, o_ref):
    i = pl.program_id(0)
    y = jnp.dot(x_ref[...], wl_ref[...], preferred_element_type=f32)
    m = (b_ref[...] == lax.broadcasted_iota(i32, (x_ref.shape[0], NG), 1))
    part = lax.dot_general(m.astype(f32), y, (((0,), (0,)), ((), ())),
                           preferred_element_type=f32)

    @pl.when(i == 0)
    def _():
        o_ref[...] = bl_ref[...] + part

    @pl.when(i != 0)
    def _():
        o_ref[...] += part


def _sds(shape):
    return jax.ShapeDtypeStruct(shape, f32)


def _dinv_dense(cnt):
    nblk = S_DEG // Bp
    return pl.pallas_call(
        _dinv_body,
        grid=(nblk,),
        in_specs=[pl.BlockSpec((2, Bp, DEGW), lambda i: (0, i, 0))],
        out_specs=pl.BlockSpec((Bp, 64), lambda i: (i, 0)),
        out_shape=_sds((S_DEG, 64)),
    )(cnt)


def _mm(xmat, w):
    return pl.pallas_call(_mm_body, out_shape=_sds((xmat.shape[0], 64)))(xmat, w)


def _prep_scaled(xblk, w, sp, d):
    return pl.pallas_call(_prep_scaled_body, out_shape=_sds((Bp, 64)))(xblk, w, sp, d)


def _prep_plain(xblk, w, sp):
    return pl.pallas_call(_prep_plain_body, out_shape=_sds((Bp, 64)))(xblk, w, sp)


def _prep_pair(xa, xb, w, spa, spb):
    return pl.pallas_call(
        _prep_pair_body, out_shape=[_sds((Bp, 64))] * 2)(xa, xb, w, spa, spb)


def _fin_inner_blk(g, spart, d, bvec):
    return pl.pallas_call(_fin_inner_body, out_shape=_sds((Bp, 64)))(g, spart, d, bvec)


def _fin_cross_blk(h, spart, d, bvec):
    return pl.pallas_call(_fin_cross_body, out_shape=_sds((Bp, 64)))(h, spart, d, bvec)


def _prep_main(xfull, w, d):
    return pl.pallas_call(
        _prep_scaled_nosp_body,
        grid=(L,),
        in_specs=[pl.BlockSpec((Bp, 64), lambda i: (i, 0)),
                  pl.BlockSpec((64, 64), lambda i: (0, 0)),
                  pl.BlockSpec((Bp, 64), lambda i: (i, 0))],
        out_specs=pl.BlockSpec((Bp, 64), lambda i: (i, 0)),
        out_shape=_sds((Np, 64)),
    )(xfull, w, d)


def _fin_main(g, spart, d, bvec):
    R = Np // 32
    return pl.pallas_call(
        _fin_inner_body,
        grid=(32,),
        in_specs=[pl.BlockSpec((R, 64), lambda i: (i, 0)),
                  pl.BlockSpec((2, 4, R, 16), lambda i: (0, 0, i, 0)),
                  pl.BlockSpec((R, 64), lambda i: (i, 0)),
                  pl.BlockSpec((1, 64), lambda i: (0, 0))],
        out_specs=pl.BlockSpec((R, 64), lambda i: (i, 0)),
        out_shape=_sds((Np, 64)),
    )(g, spart, d, bvec)


def _relu4(blocks):
    outs = pl.pallas_call(
        _relu4_body,
        out_shape=[_sds((Bp, 64))] * 4,
    )(*blocks)
    return list(outs)


def _pool(xfull, batch_p, wl, bl):
    R = Np // 8
    return pl.pallas_call(
        _pool_body,
        grid=(8,),
        in_specs=[pl.BlockSpec((R, 64), lambda i: (i, 0)),
                  pl.BlockSpec((R, 1), lambda i: (i, 0)),
                  pl.BlockSpec((64, 1), lambda i: (0, 0)),
                  pl.BlockSpec((1, 1), lambda i: (0, 0))],
        out_specs=pl.BlockSpec((NG, 1), lambda i: (0, 0)),
        out_shape=_sds((NG, 1)),
    )(xfull, batch_p, wl, bl)


def kernel(x, feature_mtx_static, edge_index, inner_edges, forward_edges,
           backward_edges, batch, W_up, b_up, W_in, b_in, W_fw, b_fw,
           W_bw, b_bw, W_lin, b_lin):
    def padrows(a):
        return jnp.pad(a, ((0, Bp - B), (0, 0)))

    xb = [padrows(x[l * B:(l + 1) * B]) for l in range(L)]
    stp = jnp.concatenate(
        [padrows(feature_mtx_static[l * B:(l + 1) * B]) for l in range(L)], 0)

    remap = lambda v: (v + 44 * (v // B)).astype(i32)
    src_m = remap(edge_index[0])
    dst_m = remap(edge_index[1])

    CHP = 128
    C_IN = 49
    C_X = 25
    C_M = 200
    C_D = 576

    in_s = [_shard_idx(inner_edges[l, 0], 0, C_IN, CHP) for l in range(L)]
    in_d = [_shard_idx(inner_edges[l, 1], Bp - 1, C_IN, CHP) for l in range(L)]
    fw_s = [_shard_idx(forward_edges[l, 0], 0, C_X, CHP) for l in range(L - 1)]
    fw_d = [_shard_idx(forward_edges[l, 1], Bp - 1, C_X, CHP) for l in range(L - 1)]
    bw_s = [_shard_idx(backward_edges[l, 0], 0, C_X, CHP) for l in range(L - 1)]
    bw_d = [_shard_idx(backward_edges[l, 1], Bp - 1, C_X, CHP) for l in range(L - 1)]
    m_s = _shard_idx(src_m, 0, C_M, CHP)
    m_d = _shard_idx(dst_m, Np - 1, C_M, CHP)

    off_in = [Np + l * Bp for l in range(L)]
    off_fw = [Np + (4 + l) * Bp for l in range(L - 1)]
    off_bw = [Np + (7 + l) * Bp for l in range(L - 1)]
    deg_dst = jnp.concatenate(
        [dst_m]
        + [inner_edges[l, 1].astype(i32) + off_in[l] for l in range(L)]
        + [forward_edges[l, 1].astype(i32) + off_fw[l] for l in range(L - 1)]
        + [backward_edges[l, 1].astype(i32) + off_bw[l] for l in range(L - 1)])
    deg_sh = _shard_idx(deg_dst, S_DEG - 1, C_D, CHP)

    zeros_blk = jnp.zeros((CH, 64), f32)
    zeros_m16 = jnp.zeros((CH, 16), f32)
    zeros_deg = jnp.zeros((1024, DEGW), f32)
    ones_deg = jnp.ones((CHP, DEGW), f32)
    bu = b_up.reshape(1, 64)
    bi = b_in.reshape(1, 64)
    bf = b_fw.reshape(1, 64)
    bb = b_bw.reshape(1, 64)
    bl = b_lin.reshape(1, 1)

    cnt = _make_degree(9, 64, CHP)(deg_sh, ones_deg, zeros_deg)
    dall = _dinv_dense(cnt)
    d_m = dall[0:Np]
    d_in = [dall[off_in[l]:off_in[l] + Bp] for l in range(L)]
    d_fw = [dall[off_fw[l]:off_fw[l] + Bp] for l in range(L - 1)]
    d_bw = [dall[off_bw[l]:off_bw[l] + Bp] for l in range(L - 1)]

    sp_in = _mm(stp, W_in[64:])
    sp_fw = _mm(stp, W_fw[64:])
    sp_bw = _mm(stp, W_bw[64:])
    spb_in = [sp_in[l * Bp:(l + 1) * Bp] for l in range(L)]
    spb_fw = [sp_fw[l * Bp:(l + 1) * Bp] for l in range(L)]
    spb_bw = [sp_bw[l * Bp:(l + 1) * Bp] for l in range(L)]
    Wd_in, Wd_fw, Wd_bw = W_in[:64], W_fw[:64], W_bw[:64]

    scat_blk = _make_scatter(Bp, 1, C_IN, 1, CHP)
    scat_x = _make_scatter(Bp, 1, C_X, 1, CHP)
    scat_main = _make_scatter(Np, 5, 40, 4, CHP)

    xfull = jnp.concatenate(xb, 0)
    g = _prep_main(xfull, W_up, d_m)
    s = scat_main(g[:, 0:16], g[:, 16:32], g[:, 32:48], g[:, 48:64],
                  m_s, m_d, zeros_m16)
    xfull = _fin_main(g, s, d_m, bu)
    xb = [xfull[l * Bp:(l + 1) * Bp] for l in range(L)]

    def inner(xb, l):
        g = _prep_scaled(xb[l], Wd_in, spb_in[l], d_in[l])
        spart = scat_blk(g, in_s[l], in_d[l], zeros_blk)
        xb[l] = _fin_inner_blk(g, spart, d_in[l], bi)

    def fwd(xb, l):
        h_src, h_dst = _prep_pair(xb[l], xb[l + 1], Wd_fw, spb_fw[l], spb_fw[l + 1])
        spart = scat_x(h_src, fw_s[l], fw_d[l], zeros_blk)
        xb[l + 1] = _fin_cross_blk(h_dst, spart, d_fw[l], bf)

    def bwd(xb, l):
        h_src, h_dst = _prep_pair(xb[l], xb[l - 1], Wd_bw, spb_bw[l], spb_bw[l - 1])
        spart = scat_x(h_src, bw_s[l - 1], bw_d[l - 1], zeros_blk)
        xb[l - 1] = _fin_cross_blk(h_dst, spart, d_bw[l - 1], bb)

    for _ in range(2):
        for l in range(L):
            inner(xb, l)
            if l < L - 1:
                fwd(xb, l)
        xb = _relu4(xb)
        for l in range(L - 1, 0, -1):
            bwd(xb, l)
            inner(xb, l - 1)
        xb = _relu4(xb)

    xfull = jnp.concatenate(xb, 0)
    batch_p = jnp.concatenate(
        [jnp.pad(batch[l * B:(l + 1) * B].astype(i32), (0, Bp - B),
                 constant_values=NG) for l in range(L)]).reshape(Np, 1)
    return _pool(xfull, batch_p, W_lin, bl)


# speedup vs baseline: 2.1327x; 1.0810x over previous
import functools

import jax
import jax.numpy as jnp
from jax import lax
from jax.experimental import pallas as pl
from jax.experimental.pallas import tpu as pltpu
from jax.experimental.pallas import tpu_sc as plsc

N = 50000
L = 4
B = N // L
Bp = 12544
Np = L * Bp
NG = 8
CH = 128
NW = 32
S_DEG = Np + 10 * Bp
DEGW = 8

f32 = jnp.float32
i32 = jnp.int32


def _cdiv(a, b):
    return (a + b - 1) // b


def _chunks(total, step):
    out, o = [], 0
    while o < total:
        n = min(step, total - o)
        out.append((o, n))
        o += n
    return out


@functools.lru_cache(maxsize=None)
def _make_scatter(A, nseg, cseg, npass, chp=CH):
    colw = 64 // npass
    rpt = A // 16
    mesh = plsc.VectorSubcoreMesh(core_axis_name="c", subcore_axis_name="s")

    def body(*refs):
        tables = refs[:npass]
        sidx, didx, zeros, out = refs[npass:npass + 4]
        siv, div, rv0, zb, acc, sg0 = refs[npass + 4:]
        c = lax.axis_index("c")
        s = lax.axis_index("s")
        w = c * 16 + s
        r0 = s * rpt
        pltpu.sync_copy(zeros, zb)
        for p in range(npass):
            tbl = tables[p]
            for (o, n) in _chunks(rpt, CH):
                pltpu.sync_copy(zb.at[pl.ds(0, n)], acc.at[pl.ds(r0 + o, n)])
            plsc.subcore_barrier()
            for seg in range(nseg):
                pltpu.sync_copy(sidx.at[w, pl.ds(seg * cseg, cseg)], siv)
                pltpu.sync_copy(didx.at[w, pl.ds(seg * cseg, cseg)], div)

                def step(j, carry):
                    pltpu.async_copy(tbl.at[siv.at[j]], rv0, sg0).wait()
                    pltpu.sync_copy(rv0, acc.at[div.at[j]], add=True)
                    return carry

                lax.fori_loop(0, cseg, step, 0)
            plsc.subcore_barrier()
            for (o, n) in _chunks(rpt, CH):
                pltpu.sync_copy(acc.at[pl.ds(r0 + o, n)], rv0.at[pl.ds(0, n)])
                pltpu.sync_copy(rv0.at[pl.ds(0, n)], out.at[c, p, pl.ds(r0 + o, n)])

    return pl.kernel(
        body,
        out_type=jax.ShapeDtypeStruct((2, npass, A, colw), f32),
        mesh=mesh,
        compiler_params=pltpu.CompilerParams(use_tc_tiling_on_sc=False),
        scratch_types=[
            pltpu.VMEM((cseg, chp), i32),
            pltpu.VMEM((cseg, chp), i32),
            pltpu.VMEM((chp, colw), f32),
            pltpu.VMEM((CH, colw), f32),
            pltpu.VMEM_SHARED((A, colw), f32),
            pltpu.SemaphoreType.DMA,
        ],
    )


@functools.lru_cache(maxsize=None)
def _make_degree(nseg, cseg, chp=CH):
    rpt = S_DEG // 16
    mesh = plsc.VectorSubcoreMesh(core_axis_name="c", subcore_axis_name="s")

    def body(didx, ones_h, zeros, out, div, ov, zb, acc, sd):
        c = lax.axis_index("c")
        s = lax.axis_index("s")
        w = c * 16 + s
        r0 = s * rpt
        pltpu.sync_copy(ones_h, ov)
        pltpu.sync_copy(zeros, zb)
        for (o, n) in _chunks(rpt, 1024):
            pltpu.sync_copy(zb.at[pl.ds(0, n)], acc.at[pl.ds(r0 + o, n)])
        plsc.subcore_barrier()
        for seg in range(nseg):
            pltpu.sync_copy(didx.at[w, pl.ds(seg * cseg, cseg)], div)

            def step(j, carry):
                pltpu.sync_copy(ov, acc.at[div.at[j]], add=True)
                return carry

            lax.fori_loop(0, cseg, step, 0)
        plsc.subcore_barrier()
        for (o, n) in _chunks(rpt, 1024):
            pltpu.sync_copy(acc.at[pl.ds(r0 + o, n)], zb.at[pl.ds(0, n)])
            pltpu.sync_copy(zb.at[pl.ds(0, n)], out.at[c, pl.ds(r0 + o, n)])

    return pl.kernel(
        body,
        out_type=jax.ShapeDtypeStruct((2, S_DEG, DEGW), f32),
        mesh=mesh,
        compiler_params=pltpu.CompilerParams(use_tc_tiling_on_sc=False),
        scratch_types=[
            pltpu.VMEM((cseg, chp), i32),
            pltpu.VMEM((chp, DEGW), f32),
            pltpu.VMEM((1024, DEGW), f32),
            pltpu.VMEM_SHARED((S_DEG, DEGW), f32),
            pltpu.SemaphoreType.DMA,
        ],
    )


def _shard_idx(idx, fill, C, chp=CH):
    ep = NW * C * chp
    idx = jnp.pad(idx.astype(i32), (0, ep - idx.shape[0]), constant_values=fill)
    return idx.reshape(NW, C, chp)


def _dinv_body(c_ref, o_ref):
    t = 1.0 + c_ref[0, :, 0:1] + c_ref[1, :, 0:1]
    o_ref[...] = jnp.broadcast_to(lax.rsqrt(t), o_ref.shape)


def _mm_body(x_ref, w_ref, o_ref):
    o_ref[...] = jnp.dot(x_ref[...], w_ref[...], preferred_element_type=f32)


def _prep_scaled_body(x_ref, w_ref, sp_ref, d_ref, o_ref):
    h = jnp.dot(x_ref[...], w_ref[...], preferred_element_type=f32) + sp_ref[...]
    o_ref[...] = h * d_ref[...]


def _prep_scaled_nosp_body(x_ref, w_ref, d_ref, o_ref):
    h = jnp.dot(x_ref[...], w_ref[...], preferred_element_type=f32)
    o_ref[...] = h * d_ref[...]


def _prep_plain_body(x_ref, w_ref, sp_ref, o_ref):
    o_ref[...] = jnp.dot(x_ref[...], w_ref[...], preferred_element_type=f32) + sp_ref[...]


def _prep_pair_body(xa_ref, xb_ref, w_ref, spa_ref, spb_ref, oa_ref, ob_ref):
    oa_ref[...] = jnp.dot(xa_ref[...], w_ref[...], preferred_element_type=f32) + spa_ref[...]
    ob_ref[...] = jnp.dot(xb_ref[...], w_ref[...], preferred_element_type=f32) + spb_ref[...]


def _ssum(s_ref):
    npass = s_ref.shape[1]
    parts = [s_ref[0, p] + s_ref[1, p] for p in range(npass)]
    return parts[0] if npass == 1 else jnp.concatenate(parts, axis=1)


def _fin_inner_body(g_ref, s_ref, d_ref, b_ref, o_ref):
    o_ref[...] = b_ref[...] + d_ref[...] * (g_ref[...] + _ssum(s_ref))


def _fin_cross_body(h_ref, s_ref, d_ref, b_ref, o_ref):
    d = d_ref[...]
    o_ref[...] = b_ref[...] + d * (d * h_ref[...] + _ssum(s_ref))


def _relu4_body(a_ref, b_ref, c_ref, d_ref, oa, ob, oc, od):
    oa[...] = jnp.maximum(a_ref[...], 0.0)
    ob[...] = jnp.maximum(b_ref[...], 0.0)
    oc[...] = jnp.maximum(c_ref[...], 0.0)
    od[...] = jnp.maximum(d_ref[...], 0.0)


def _pool_body(x_ref, b_ref, wl_ref, bl_ref, o_ref):
    i = pl.program_id(0)
    y = jnp.dot(x_ref[...], wl_ref[...], preferred_element_type=f32)
    m = (b_ref[...] == lax.broadcasted_iota(i32, (x_ref.shape[0], NG), 1))
    part = lax.dot_general(m.astype(f32), y, (((0,), (0,)), ((), ())),
                           preferred_element_type=f32)

    @pl.when(i == 0)
    def _():
        o_ref[...] = bl_ref[...] + part

    @pl.when(i != 0)
    def _():
        o_ref[...] += part


def _sds(shape):
    return jax.ShapeDtypeStruct(shape, f32)


def _dinv_dense(cnt):
    nblk = S_DEG // Bp
    return pl.pallas_call(
        _dinv_body,
        grid=(nblk,),
        in_specs=[pl.BlockSpec((2, Bp, DEGW), lambda i: (0, i, 0))],
        out_specs=pl.BlockSpec((Bp, 64), lambda i: (i, 0)),
        out_shape=_sds((S_DEG, 64)),
    )(cnt)


def _mm(xmat, w):
    return pl.pallas_call(_mm_body, out_shape=_sds((xmat.shape[0], 64)))(xmat, w)


def _prep_scaled(xblk, w, sp, d):
    return pl.pallas_call(_prep_scaled_body, out_shape=_sds((Bp, 64)))(xblk, w, sp, d)


def _prep_plain(xblk, w, sp):
    return pl.pallas_call(_prep_plain_body, out_shape=_sds((Bp, 64)))(xblk, w, sp)


def _prep_pair(xa, xb, w, spa, spb):
    return pl.pallas_call(
        _prep_pair_body, out_shape=[_sds((Bp, 64))] * 2)(xa, xb, w, spa, spb)


def _fin_inner_blk(g, spart, d, bvec):
    return pl.pallas_call(_fin_inner_body, out_shape=_sds((Bp, 64)))(g, spart, d, bvec)


def _fin_cross_blk(h, spart, d, bvec):
    return pl.pallas_call(_fin_cross_body, out_shape=_sds((Bp, 64)))(h, spart, d, bvec)


def _prep_main(xfull, w, d):
    return pl.pallas_call(
        _prep_scaled_nosp_body,
        grid=(L,),
        in_specs=[pl.BlockSpec((Bp, 64), lambda i: (i, 0)),
                  pl.BlockSpec((64, 64), lambda i: (0, 0)),
                  pl.BlockSpec((Bp, 64), lambda i: (i, 0))],
        out_specs=pl.BlockSpec((Bp, 64), lambda i: (i, 0)),
        out_shape=_sds((Np, 64)),
    )(xfull, w, d)


def _fin_main(g, spart, d, bvec):
    R = Np // 32
    return pl.pallas_call(
        _fin_inner_body,
        grid=(32,),
        in_specs=[pl.BlockSpec((R, 64), lambda i: (i, 0)),
                  pl.BlockSpec((2, 2, R, 32), lambda i: (0, 0, i, 0)),
                  pl.BlockSpec((R, 64), lambda i: (i, 0)),
                  pl.BlockSpec((1, 64), lambda i: (0, 0))],
        out_specs=pl.BlockSpec((R, 64), lambda i: (i, 0)),
        out_shape=_sds((Np, 64)),
    )(g, spart, d, bvec)


def _relu4(blocks):
    outs = pl.pallas_call(
        _relu4_body,
        out_shape=[_sds((Bp, 64))] * 4,
    )(*blocks)
    return list(outs)


def _pool(xfull, batch_p, wl, bl):
    R = Np // 8
    return pl.pallas_call(
        _pool_body,
        grid=(8,),
        in_specs=[pl.BlockSpec((R, 64), lambda i: (i, 0)),
                  pl.BlockSpec((R, 1), lambda i: (i, 0)),
                  pl.BlockSpec((64, 1), lambda i: (0, 0)),
                  pl.BlockSpec((1, 1), lambda i: (0, 0))],
        out_specs=pl.BlockSpec((NG, 1), lambda i: (0, 0)),
        out_shape=_sds((NG, 1)),
    )(xfull, batch_p, wl, bl)


def kernel(x, feature_mtx_static, edge_index, inner_edges, forward_edges,
           backward_edges, batch, W_up, b_up, W_in, b_in, W_fw, b_fw,
           W_bw, b_bw, W_lin, b_lin):
    def padrows(a):
        return jnp.pad(a, ((0, Bp - B), (0, 0)))

    xb = [padrows(x[l * B:(l + 1) * B]) for l in range(L)]
    stp = jnp.concatenate(
        [padrows(feature_mtx_static[l * B:(l + 1) * B]) for l in range(L)], 0)

    remap = lambda v: (v + 44 * (v // B)).astype(i32)
    src_m = remap(edge_index[0])
    dst_m = remap(edge_index[1])

    CHP = 128
    C_IN = 49
    C_X = 25
    C_M = 200
    C_D = 576

    in_s = [_shard_idx(inner_edges[l, 0], 0, C_IN, CHP) for l in range(L)]
    in_d = [_shard_idx(inner_edges[l, 1], Bp - 1, C_IN, CHP) for l in range(L)]
    fw_s = [_shard_idx(forward_edges[l, 0], 0, C_X, CHP) for l in range(L - 1)]
    fw_d = [_shard_idx(forward_edges[l, 1], Bp - 1, C_X, CHP) for l in range(L - 1)]
    bw_s = [_shard_idx(backward_edges[l, 0], 0, C_X, CHP) for l in range(L - 1)]
    bw_d = [_shard_idx(backward_edges[l, 1], Bp - 1, C_X, CHP) for l in range(L - 1)]
    m_s = _shard_idx(src_m, 0, C_M, CHP)
    m_d = _shard_idx(dst_m, Np - 1, C_M, CHP)

    off_in = [Np + l * Bp for l in range(L)]
    off_fw = [Np + (4 + l) * Bp for l in range(L - 1)]
    off_bw = [Np + (7 + l) * Bp for l in range(L - 1)]
    deg_dst = jnp.concatenate(
        [dst_m]
        + [inner_edges[l, 1].astype(i32) + off_in[l] for l in range(L)]
        + [forward_edges[l, 1].astype(i32) + off_fw[l] for l in range(L - 1)]
        + [backward_edges[l, 1].astype(i32) + off_bw[l] for l in range(L - 1)])
    deg_sh = _shard_idx(deg_dst, S_DEG - 1, C_D, CHP)

    zeros_blk = jnp.zeros((CH, 64), f32)
    zeros_m32 = jnp.zeros((CH, 32), f32)
    zeros_deg = jnp.zeros((1024, DEGW), f32)
    ones_deg = jnp.ones((CHP, DEGW), f32)
    bu = b_up.reshape(1, 64)
    bi = b_in.reshape(1, 64)
    bf = b_fw.reshape(1, 64)
    bb = b_bw.reshape(1, 64)
    bl = b_lin.reshape(1, 1)

    cnt = _make_degree(9, 64, CHP)(deg_sh, ones_deg, zeros_deg)
    dall = _dinv_dense(cnt)
    d_m = dall[0:Np]
    d_in = [dall[off_in[l]:off_in[l] + Bp] for l in range(L)]
    d_fw = [dall[off_fw[l]:off_fw[l] + Bp] for l in range(L - 1)]
    d_bw = [dall[off_bw[l]:off_bw[l] + Bp] for l in range(L - 1)]

    sp_in = _mm(stp, W_in[64:])
    sp_fw = _mm(stp, W_fw[64:])
    sp_bw = _mm(stp, W_bw[64:])
    spb_in = [sp_in[l * Bp:(l + 1) * Bp] for l in range(L)]
    spb_fw = [sp_fw[l * Bp:(l + 1) * Bp] for l in range(L)]
    spb_bw = [sp_bw[l * Bp:(l + 1) * Bp] for l in range(L)]
    Wd_in, Wd_fw, Wd_bw = W_in[:64], W_fw[:64], W_bw[:64]

    scat_blk = _make_scatter(Bp, 1, C_IN, 1, CHP)
    scat_x = _make_scatter(Bp, 1, C_X, 1, CHP)
    scat_main = _make_scatter(Np, 5, 40, 2, CHP)

    xfull = jnp.concatenate(xb, 0)
    g = _prep_main(xfull, W_up, d_m)
    s = scat_main(g[:, 0:32], g[:, 32:64], m_s, m_d, zeros_m32)
    xfull = _fin_main(g, s, d_m, bu)
    xb = [xfull[l * Bp:(l + 1) * Bp] for l in range(L)]

    def inner(xb, l):
        g = _prep_scaled(xb[l], Wd_in, spb_in[l], d_in[l])
        spart = scat_blk(g, in_s[l], in_d[l], zeros_blk)
        xb[l] = _fin_inner_blk(g, spart, d_in[l], bi)

    def fwd(xb, l):
        h_src, h_dst = _prep_pair(xb[l], xb[l + 1], Wd_fw, spb_fw[l], spb_fw[l + 1])
        spart = scat_x(h_src, fw_s[l], fw_d[l], zeros_blk)
        xb[l + 1] = _fin_cross_blk(h_dst, spart, d_fw[l], bf)

    def bwd(xb, l):
        h_src, h_dst = _prep_pair(xb[l], xb[l - 1], Wd_bw, spb_bw[l], spb_bw[l - 1])
        spart = scat_x(h_src, bw_s[l - 1], bw_d[l - 1], zeros_blk)
        xb[l - 1] = _fin_cross_blk(h_dst, spart, d_bw[l - 1], bb)

    for _ in range(2):
        for l in range(L):
            inner(xb, l)
            if l < L - 1:
                fwd(xb, l)
        xb = _relu4(xb)
        for l in range(L - 1, 0, -1):
            bwd(xb, l)
            inner(xb, l - 1)
        xb = _relu4(xb)

    xfull = jnp.concatenate(xb, 0)
    batch_p = jnp.concatenate(
        [jnp.pad(batch[l * B:(l + 1) * B].astype(i32), (0, Bp - B),
                 constant_values=NG) for l in range(L)]).reshape(Np, 1)
    return _pool(xfull, batch_p, W_lin, bl)
